# Initial kernel scaffold; baseline (speedup 1.0000x reference)
import jax
import jax.numpy as jnp
from jax.experimental import pallas as pl


def kernel(x, edge_index, edge_attr, pg_min, pg_max, gen_bus_idx, gen_indices, W_emb, b_emb, Wm, bm, Wu, bu, gamma, beta, W1, b1, W2, b2):
    raise NotImplementedError("write your pallas kernel here")


# speedup vs baseline: 1.1175x; 1.1175x over previous
import functools

import jax
import jax.numpy as jnp
from jax.experimental import pallas as pl
from jax.experimental.pallas import tpu as pltpu

BB = 32


def _split2(v):
    hi = v.astype(jnp.bfloat16)
    lo = (v - hi.astype(jnp.float32)).astype(jnp.bfloat16)
    return hi, lo


def _tanh(x):
    x = jnp.clip(x, -4.97, 4.97)
    x2 = x * x
    num = x * (135135.0 + x2 * (17325.0 + x2 * (378.0 + x2)))
    den = 135135.0 + x2 * (62370.0 + x2 * (3150.0 + x2 * 28.0))
    return num / den


def _dotp(a, w):
    ahi, alo = _split2(a)
    whi, wlo = _split2(w)
    ap = jnp.concatenate([ahi, ahi, alo], axis=-1)
    wp = jnp.concatenate([whi, wlo, whi], axis=0)
    return jnp.dot(ap, wp, preferred_element_type=jnp.float32)


def _gnn_body(xC_ref, A_ref, M_ref, deginv_ref, nbias_ref, w0_ref, ea_ref,
              Wmh_ref, Wmet_ref, bmt_ref, Wu_ref, bu_ref, g_ref, b_ref,
              W1_ref, b1_ref, W2_ref, b2_ref, out_ref, scrB, scrA,
              *, Np, E, H, L):
    f32 = jnp.float32

    x2 = xC_ref[0]
    nb = nbias_ref[...]
    w0 = w0_ref[...]
    h = _tanh(x2[:, :, None] * w0[None, :, :] + nb[None, :, :])
    h2 = h.reshape(BB * Np, H)

    A = A_ref[...]
    M = M_ref[...]
    deginv = deginv_ref[...]
    ea = ea_ref[...]

    for l in range(L):
        hW = _dotp(h2, Wmh_ref[l])
        for b in range(BB):
            scrB[:, b * H:(b + 1) * H] = jax.lax.slice(
                hW, (b * Np, 0), ((b + 1) * Np, H))
        hWn = scrB[...]
        hi, lo = _split2(hWn)
        t = (jnp.dot(A, hi, preferred_element_type=f32)
             + jnp.dot(A, lo, preferred_element_type=f32))
        ebig = _dotp(ea, Wmet_ref[l]) + bmt_ref[l:l + 1, :]
        msg = _tanh(t + ebig)
        mhi, mlo = _split2(msg)
        agg = (jnp.dot(M, mhi, preferred_element_type=f32)
               + jnp.dot(M, mlo, preferred_element_type=f32)) * deginv
        for b in range(BB):
            scrA[b * Np:(b + 1) * Np, :] = jax.lax.slice(
                agg, (0, b * H), (Np, (b + 1) * H))
        agg2 = scrA[...]
        cat = jnp.concatenate([h2, agg2], axis=-1)
        u = _tanh(_dotp(cat, Wu_ref[l]) + bu_ref[l:l + 1, :])
        mu = jnp.mean(u, axis=-1, keepdims=True)
        var = jnp.mean((u - mu) ** 2, axis=-1, keepdims=True) + 1e-5
        r = jax.lax.rsqrt(var)
        r = r * (1.5 - 0.5 * var * r * r)
        h2 = g_ref[l:l + 1, :] * (u - mu) * r + b_ref[l:l + 1, :]

    z = _tanh(_dotp(h2, W1_ref[...]) + b1_ref[...])
    pg = _dotp(z, W2_ref[...]) + b2_ref[...]
    out_ref[0] = pg


def kernel(x, edge_index, edge_attr, pg_min, pg_max, gen_bus_idx, gen_indices,
           W_emb, b_emb, Wm, bm, Wu, bu, gamma, beta, W1, b1, W2, b2):
    B, N = x.shape
    E = edge_index.shape[1]
    L, _, H = Wm.shape
    Np = ((N + 7) // 8) * 8
    bf16 = jnp.bfloat16

    src = edge_index[0]
    dst = edge_index[1]
    nids = jnp.arange(Np, dtype=src.dtype)
    A = (src[:, None] == nids[None, :]).astype(bf16)
    M = (dst[None, :] == nids[:, None]).astype(bf16)
    deg = jnp.maximum(jnp.zeros((Np,), jnp.float32).at[dst].add(1.0), 1.0)
    deginv = (1.0 / deg)[:, None]

    pmin = jnp.zeros((Np,), x.dtype).at[gen_bus_idx].set(pg_min)
    pmax = jnp.zeros((Np,), x.dtype).at[gen_bus_idx].set(pg_max)
    gmask = jnp.zeros((Np,), x.dtype).at[gen_bus_idx].set(1.0)
    nbias = (pmin[:, None] * W_emb[1][None, :]
             + pmax[:, None] * W_emb[2][None, :]
             + gmask[:, None] * W_emb[3][None, :]
             + W_emb[4][None, :] + b_emb[None, :])
    w0 = W_emb[0][None, :]

    xC = jnp.pad(x, ((0, 0), (0, Np - N))).reshape(B // BB, BB, Np)
    Wmh = Wm[:, :H, :]
    Wmet = jnp.tile(Wm[:, H:, :], (1, 1, BB))
    bmt = jnp.tile(bm, (1, BB))

    grid = (B // BB,)
    full = lambda s: pl.BlockSpec(s, lambda i: (0,) * len(s))
    body = functools.partial(_gnn_body, Np=Np, E=E, H=H, L=L)
    pg_full = pl.pallas_call(
        body,
        grid=grid,
        in_specs=[
            pl.BlockSpec((1, BB, Np), lambda i: (i, 0, 0)),
            full((E, Np)), full((Np, E)), full((Np, 1)), full((Np, H)),
            full((1, H)), full((E, 2)),
            full((L, H, H)), full((L, 2, BB * H)), full((L, BB * H)),
            full((L, 2 * H, H)), full((L, H)), full((L, H)), full((L, H)),
            full((H, H // 2)), full((1, H // 2)),
            full((H // 2, 1)), full((1, 1)),
        ],
        out_specs=pl.BlockSpec((1, BB * Np, 1), lambda i: (i, 0, 0)),
        out_shape=jax.ShapeDtypeStruct((B // BB, BB * Np, 1), jnp.float32),
        scratch_shapes=[
            pltpu.VMEM((Np, BB * H), jnp.float32),
            pltpu.VMEM((BB * Np, H), jnp.float32),
        ],
    )(xC, A, M, deginv, nbias, w0, edge_attr,
      Wmh, Wmet, bmt, Wu, bu, gamma, beta,
      W1, b1[None, :], W2, b2[None, :])

    pg_bn = pg_full.reshape(B, Np)
    return pg_bn[:, gen_indices]
